# Initial kernel scaffold; baseline (speedup 1.0000x reference)
import jax
import jax.numpy as jnp
from jax.experimental import pallas as pl


def kernel(coords, keys):
    raise NotImplementedError("write your pallas kernel here")


# speedup vs baseline: 1.6019x; 1.6019x over previous
import jax
import jax.numpy as jnp
from jax.experimental import pallas as pl
from jax.experimental.pallas import tpu as pltpu

_Q = 1024
_D = 16
_K = 100000
_BK = 2048
_KP = 100352
_NB = _KP // _BK
_RS = 64
_NS = _Q // _RS
_NC = _BK // 128


def _knn_kernel(q2_ref, kt_ref, out_ref, qsqb_ref, dot_ref, minval, minblk, minloc):
    kb = pl.program_id(0)

    @pl.when(kb == 0)
    def _():
        q = q2_ref[...] * 0.5
        qsq = jnp.sum(q * q, axis=1, keepdims=True)
        qsqb_ref[...] = jnp.broadcast_to(qsq, (_Q, 128))
        minval[...] = jnp.full((_Q, 1), 3.0e38, jnp.float32)
        minblk[...] = jnp.zeros((_Q, 1), jnp.int32)
        minloc[...] = jnp.zeros((_Q, 1), jnp.int32)

    kt = kt_ref[...]
    ksq = jnp.sum(kt * kt, axis=0, keepdims=True)
    dot_ref[...] = jnp.dot(q2_ref[...], kt, preferred_element_type=jnp.float32)

    for s in range(_NS):
        rs = slice(s * _RS, (s + 1) * _RS)
        qb = qsqb_ref[rs, :]
        m = (qb + ksq[:, 0:128]) - dot_ref[rs, 0:128]
        cid = jnp.zeros((_RS, 128), jnp.int32)
        for c in range(1, _NC):
            d2c = (qb + ksq[:, c * 128:(c + 1) * 128]) - dot_ref[rs, c * 128:(c + 1) * 128]
            lt = d2c < m
            m = jnp.where(lt, d2c, m)
            cid = jnp.where(lt, c, cid)
        tmin = jnp.min(m, axis=1, keepdims=True)
        lane = jax.lax.broadcasted_iota(jnp.int32, (_RS, 128), 1)
        j = cid * 128 + lane
        tloc = jnp.min(jnp.where(m == tmin, j, jnp.int32(2**30)),
                       axis=1, keepdims=True)
        mv = minval[rs, :]
        better = tmin < mv
        minblk[rs, :] = jnp.where(better, kb, minblk[rs, :])
        minloc[rs, :] = jnp.where(better, tloc, minloc[rs, :])
        minval[rs, :] = jnp.where(better, tmin, mv)

    @pl.when(kb == _NB - 1)
    def _():
        out_ref[...] = minblk[...] * _BK + minloc[...]


def kernel(coords, keys):
    kt = jnp.pad(keys, ((0, _KP - _K), (0, 0)), constant_values=1000.0).T
    q2 = coords * 2.0
    out = pl.pallas_call(
        _knn_kernel,
        grid=(_NB,),
        in_specs=[
            pl.BlockSpec((_Q, _D), lambda kb: (0, 0)),
            pl.BlockSpec((_D, _BK), lambda kb: (0, kb)),
        ],
        out_specs=pl.BlockSpec((_Q, 1), lambda kb: (0, 0)),
        out_shape=jax.ShapeDtypeStruct((_Q, 1), jnp.int32),
        scratch_shapes=[
            pltpu.VMEM((_Q, 128), jnp.float32),
            pltpu.VMEM((_Q, _BK), jnp.float32),
            pltpu.VMEM((_Q, 1), jnp.float32),
            pltpu.VMEM((_Q, 1), jnp.int32),
            pltpu.VMEM((_Q, 1), jnp.int32),
        ],
    )(q2, kt)
    return out[:, 0]


# speedup vs baseline: 1.7851x; 1.1144x over previous
import jax
import jax.numpy as jnp
from jax.experimental import pallas as pl
from jax.experimental.pallas import tpu as pltpu

_Q = 1024
_D = 16
_K = 100000
_BK = 4096
_KP = 102400
_NB = _KP // _BK
_RS = 64
_NS = _Q // _RS
_NC = _BK // 128


def _knn_kernel(q2_ref, kt_ref, out_ref, qsqb_ref, dot_ref, minval, minblk, minloc):
    kb = pl.program_id(0)

    @pl.when(kb == 0)
    def _():
        q = q2_ref[...] * 0.5
        qsq = jnp.sum(q * q, axis=1, keepdims=True)
        qsqb_ref[...] = jnp.broadcast_to(qsq, (_Q, 128))
        minval[...] = jnp.full((_Q, 1), 3.0e38, jnp.float32)
        minblk[...] = jnp.zeros((_Q, 1), jnp.int32)
        minloc[...] = jnp.zeros((_Q, 1), jnp.int32)

    kt = kt_ref[...]
    ksq = jnp.sum(kt * kt, axis=0, keepdims=True)
    dot_ref[...] = jnp.dot(q2_ref[...], kt, preferred_element_type=jnp.float32)

    for s in range(_NS):
        rs = slice(s * _RS, (s + 1) * _RS)
        qb = qsqb_ref[rs, :]
        m = (qb + ksq[:, 0:128]) - dot_ref[rs, 0:128]
        cid = jnp.zeros((_RS, 128), jnp.int32)
        for c in range(1, _NC):
            d2c = (qb + ksq[:, c * 128:(c + 1) * 128]) - dot_ref[rs, c * 128:(c + 1) * 128]
            lt = d2c < m
            m = jnp.where(lt, d2c, m)
            cid = jnp.where(lt, c, cid)
        tmin = jnp.min(m, axis=1, keepdims=True)
        lane = jax.lax.broadcasted_iota(jnp.int32, (_RS, 128), 1)
        j = cid * 128 + lane
        tloc = jnp.min(jnp.where(m == tmin, j, jnp.int32(2**30)),
                       axis=1, keepdims=True)
        mv = minval[rs, :]
        better = tmin < mv
        minblk[rs, :] = jnp.where(better, kb, minblk[rs, :])
        minloc[rs, :] = jnp.where(better, tloc, minloc[rs, :])
        minval[rs, :] = jnp.where(better, tmin, mv)

    @pl.when(kb == _NB - 1)
    def _():
        out_ref[...] = minblk[...] * _BK + minloc[...]


def kernel(coords, keys):
    kt = jnp.pad(keys, ((0, _KP - _K), (0, 0)), constant_values=1000.0).T
    q2 = coords * 2.0
    out = pl.pallas_call(
        _knn_kernel,
        grid=(_NB,),
        in_specs=[
            pl.BlockSpec((_Q, _D), lambda kb: (0, 0)),
            pl.BlockSpec((_D, _BK), lambda kb: (0, kb)),
        ],
        out_specs=pl.BlockSpec((_Q, 1), lambda kb: (0, 0)),
        out_shape=jax.ShapeDtypeStruct((_Q, 1), jnp.int32),
        scratch_shapes=[
            pltpu.VMEM((_Q, 128), jnp.float32),
            pltpu.VMEM((_Q, _BK), jnp.float32),
            pltpu.VMEM((_Q, 1), jnp.float32),
            pltpu.VMEM((_Q, 1), jnp.int32),
            pltpu.VMEM((_Q, 1), jnp.int32),
        ],
    )(q2, kt)
    return out[:, 0]
